# baseline (speedup 1.0000x reference)
import jax
import jax.numpy as jnp
from jax import lax
from jax.experimental import pallas as pl
from jax.experimental.pallas import tpu as pltpu
from jax.experimental.pallas import tpu_sc as plsc

BATCH = 16384
EMBED_DIM = 32
LANES = 16
NUM_WORKERS = 32
B_PER_W = BATCH // NUM_WORKERS
IDX_CHUNK = 128
N_CHUNKS = B_PER_W // IDX_CHUNK
GROUPS = B_PER_W // LANES
DRAIN_LAG = 2


def _body(uidx_hbm, pidx_hbm, utab_t, ptab_t, out_hbm,
          uidx_v, pidx_v, ut_v, pt_v, out_v, sem):
    cid = lax.axis_index("c")
    sid = lax.axis_index("s")
    wid = sid * 2 + cid

    pltpu.sync_copy(uidx_hbm.at[pl.ds(wid * N_CHUNKS, N_CHUNKS)], uidx_v)
    pltpu.sync_copy(pidx_hbm.at[pl.ds(wid * N_CHUNKS, N_CHUNKS)], pidx_v)

    def drain(d):
        pltpu.make_async_copy(
            utab_t.at[0, pl.ds(0, B_PER_W)], ut_v.at[d], sem).wait()
        pltpu.make_async_copy(
            ptab_t.at[0, pl.ds(0, B_PER_W)], pt_v.at[d], sem).wait()

    def enqueue(d, _):
        for c in range(N_CHUNKS):
            pltpu.async_copy(
                utab_t.at[d].at[uidx_v.at[c]],
                ut_v.at[d, pl.ds(c * IDX_CHUNK, IDX_CHUNK)], sem)
            pltpu.async_copy(
                ptab_t.at[d].at[pidx_v.at[c]],
                pt_v.at[d, pl.ds(c * IDX_CHUNK, IDX_CHUNK)], sem)

        @pl.when(d >= DRAIN_LAG)
        def _():
            drain(d - DRAIN_LAG)

        return 0

    lax.fori_loop(0, EMBED_DIM, enqueue, 0)

    def tail(d, _):
        drain(d)
        return 0

    lax.fori_loop(EMBED_DIM - DRAIN_LAG, EMBED_DIM, tail, 0)

    def group(g, _):
        acc = jnp.zeros((16,), jnp.float32)
        for d in range(EMBED_DIM):
            acc = acc + (ut_v[d, pl.ds(g * LANES, LANES)]
                         * pt_v[d, pl.ds(g * LANES, LANES)])
        out_v[pl.ds(g * LANES, LANES)] = acc
        return 0

    lax.fori_loop(0, GROUPS, group, 0)

    pltpu.sync_copy(out_v, out_hbm.at[pl.ds(wid * B_PER_W, B_PER_W)])


@jax.jit
def _sc_dot(uidx, pidx, utab_t, ptab_t):
    mesh = plsc.VectorSubcoreMesh(core_axis_name="c", subcore_axis_name="s")
    kern = pl.kernel(
        _body,
        out_type=jax.ShapeDtypeStruct((BATCH,), jnp.float32),
        mesh=mesh,
        scratch_types=[
            pltpu.VMEM((N_CHUNKS, IDX_CHUNK), jnp.int32),
            pltpu.VMEM((N_CHUNKS, IDX_CHUNK), jnp.int32),
            pltpu.VMEM((EMBED_DIM, B_PER_W), jnp.float32),
            pltpu.VMEM((EMBED_DIM, B_PER_W), jnp.float32),
            pltpu.VMEM((B_PER_W,), jnp.float32),
            pltpu.SemaphoreType.DMA,
        ],
        compiler_params=pltpu.CompilerParams(
            needs_layout_passes=False, use_tc_tiling_on_sc=False),
    )
    return kern(uidx, pidx, utab_t, ptab_t)


def kernel(inputs, user_table, place_table):
    uidx = inputs[:, 0].astype(jnp.int32).reshape(-1, IDX_CHUNK)
    pidx = inputs[:, 1].astype(jnp.int32).reshape(-1, IDX_CHUNK)
    return _sc_dot(uidx, pidx, user_table.T, place_table.T)


# speedup vs baseline: 20.8007x; 20.8007x over previous
import jax
import jax.numpy as jnp
from jax import lax
from jax.experimental import pallas as pl
from jax.experimental.pallas import tpu as pltpu
from jax.experimental.pallas import tpu_sc as plsc

BATCH = 16384
EMBED_DIM = 32
LANES = 16
NUM_WORKERS = 32
B_PER_W = BATCH // NUM_WORKERS
GROUPS = B_PER_W // LANES
NSLOT = 4
VBLK = 128


def _body(uidx_hbm, pidx_hbm, utab4, ptab4, out_hbm,
          uidx_v, pidx_v, slots, ut_v, pt_v, out_v, sems):
    cid = lax.axis_index("c")
    sid = lax.axis_index("s")
    wid = sid * 2 + cid
    base = wid * B_PER_W

    pltpu.sync_copy(uidx_hbm.at[pl.ds(base, B_PER_W)], uidx_v)
    pltpu.sync_copy(pidx_hbm.at[pl.ds(base, B_PER_W)], pidx_v)

    lane = lax.iota(jnp.int32, 16)
    s_lo = lax.shift_right_logical(lane, 3)
    r_lo = lax.bitwise_and(lane, jnp.int32(7))
    s_hi = s_lo + 2
    d_lo = lane
    d_hi = lane + 16

    def enq(k, v):
        vb = lax.shift_right_logical(v, 7)
        off = pl.multiple_of(vb * VBLK, VBLK)
        pltpu.async_copy(utab4.at[:, :, pl.ds(off, VBLK)],
                         slots.at[k, 0], sems[k])

    def enq_p(k, v):
        vb = lax.shift_right_logical(v, 7)
        off = pl.multiple_of(vb * VBLK, VBLK)
        pltpu.async_copy(ptab4.at[:, :, pl.ds(off, VBLK)],
                         slots.at[k, 1], sems[k])

    def extract(k, uv, pv, i):
        pltpu.make_async_copy(
            utab4.at[:, :, pl.ds(0, VBLK)], slots.at[k, 0], sems[k]).wait()
        pltpu.make_async_copy(
            utab4.at[:, :, pl.ds(0, VBLK)], slots.at[k, 1], sems[k]).wait()
        ucol = jnp.broadcast_to(lax.bitwise_and(uv, jnp.int32(127)), (16,))
        pcol = jnp.broadcast_to(lax.bitwise_and(pv, jnp.int32(127)), (16,))
        icol = jnp.broadcast_to(i, (16,))
        u0 = plsc.load_gather(slots.at[k, 0], [s_lo, r_lo, ucol])
        u1 = plsc.load_gather(slots.at[k, 0], [s_hi, r_lo, ucol])
        p0 = plsc.load_gather(slots.at[k, 1], [s_lo, r_lo, pcol])
        p1 = plsc.load_gather(slots.at[k, 1], [s_hi, r_lo, pcol])
        plsc.store_scatter(ut_v, [d_lo, icol], u0)
        plsc.store_scatter(ut_v, [d_hi, icol], u1)
        plsc.store_scatter(pt_v, [d_lo, icol], p0)
        plsc.store_scatter(pt_v, [d_hi, icol], p1)

    def group(g, _):
        uvec = uidx_v[pl.ds(g * LANES, LANES)]
        pvec = pidx_v[pl.ds(g * LANES, LANES)]
        gm1 = lax.max(g - 1, 0)
        uvec_p = uidx_v[pl.ds(gm1 * LANES, LANES)]
        pvec_p = pidx_v[pl.ds(gm1 * LANES, LANES)]
        for j in range(LANES):
            k = j % NSLOT
            if j < NSLOT:
                @pl.when(g > 0)
                def _(j=j, k=k):
                    extract(k, uvec_p[12 + j], pvec_p[12 + j],
                            (g - 1) * LANES + 12 + j)
            else:
                extract(k, uvec[j - NSLOT], pvec[j - NSLOT],
                        g * LANES + j - NSLOT)
            enq(k, uvec[j])
            enq_p(k, pvec[j])
        return 0

    lax.fori_loop(0, GROUPS, group, 0)

    uvec_t = uidx_v[pl.ds((GROUPS - 1) * LANES, LANES)]
    pvec_t = pidx_v[pl.ds((GROUPS - 1) * LANES, LANES)]
    for j in range(NSLOT):
        extract(j % NSLOT, uvec_t[12 + j], pvec_t[12 + j],
                (GROUPS - 1) * LANES + 12 + j)

    def dot(g, _):
        acc = jnp.zeros((16,), jnp.float32)
        for d in range(EMBED_DIM):
            acc = acc + (ut_v[d, pl.ds(g * LANES, LANES)]
                         * pt_v[d, pl.ds(g * LANES, LANES)])
        out_v[pl.ds(g * LANES, LANES)] = acc
        return 0

    lax.fori_loop(0, GROUPS, dot, 0)

    pltpu.sync_copy(out_v, out_hbm.at[pl.ds(base, B_PER_W)])


@jax.jit
def _sc_dot(uidx, pidx, utab4, ptab4):
    mesh = plsc.VectorSubcoreMesh(core_axis_name="c", subcore_axis_name="s")
    kern = pl.kernel(
        _body,
        out_type=jax.ShapeDtypeStruct((BATCH,), jnp.float32),
        mesh=mesh,
        scratch_types=[
            pltpu.VMEM((B_PER_W,), jnp.int32),
            pltpu.VMEM((B_PER_W,), jnp.int32),
            pltpu.VMEM((NSLOT, 2, 4, 8, VBLK), jnp.float32),
            pltpu.VMEM((EMBED_DIM, B_PER_W), jnp.float32),
            pltpu.VMEM((EMBED_DIM, B_PER_W), jnp.float32),
            pltpu.VMEM((B_PER_W,), jnp.float32),
            [pltpu.SemaphoreType.DMA] * NSLOT,
        ],
        compiler_params=pltpu.CompilerParams(
            needs_layout_passes=False, use_tc_tiling_on_sc=True),
    )
    return kern(uidx, pidx, utab4, ptab4)


def kernel(inputs, user_table, place_table):
    uidx = inputs[:, 0].astype(jnp.int32)
    pidx = inputs[:, 1].astype(jnp.int32)
    ut4 = user_table.T.reshape(4, 8, user_table.shape[0])
    pt4 = place_table.T.reshape(4, 8, place_table.shape[0])
    return _sc_dot(uidx, pidx, ut4, pt4)


# speedup vs baseline: 24.1833x; 1.1626x over previous
import jax
import jax.numpy as jnp
from jax import lax
from jax.experimental import pallas as pl
from jax.experimental.pallas import tpu as pltpu
from jax.experimental.pallas import tpu_sc as plsc

BATCH = 16384
EMBED_DIM = 32
LANES = 16
NUM_WORKERS = 32
B_PER_W = BATCH // NUM_WORKERS
GROUPS = B_PER_W // LANES
NSLOT = 8
VBLK = 128


def _body(uidx_hbm, pidx_hbm, utab4, ptab4, out_hbm,
          uidx_v, pidx_v, slots, ut_v, pt_v, out_v, sems):
    cid = lax.axis_index("c")
    sid = lax.axis_index("s")
    wid = sid * 2 + cid
    base = wid * B_PER_W

    pltpu.sync_copy(uidx_hbm.at[pl.ds(base, B_PER_W)], uidx_v)
    pltpu.sync_copy(pidx_hbm.at[pl.ds(base, B_PER_W)], pidx_v)

    lane = lax.iota(jnp.int32, 16)
    s_lo = lax.shift_right_logical(lane, 3)
    r_lo = lax.bitwise_and(lane, jnp.int32(7))
    s_hi = s_lo + 2
    d_lo = lane
    d_hi = lane + 16

    def enq(k, v):
        vb = lax.shift_right_logical(v, 7)
        off = pl.multiple_of(vb * VBLK, VBLK)
        pltpu.async_copy(utab4.at[:, :, pl.ds(off, VBLK)],
                         slots.at[k, 0], sems[k])

    def enq_p(k, v):
        vb = lax.shift_right_logical(v, 7)
        off = pl.multiple_of(vb * VBLK, VBLK)
        pltpu.async_copy(ptab4.at[:, :, pl.ds(off, VBLK)],
                         slots.at[k, 1], sems[k])

    def extract(k, uv, pv, i):
        pltpu.make_async_copy(
            utab4.at[:, :, pl.ds(0, VBLK)], slots.at[k, 0], sems[k]).wait()
        pltpu.make_async_copy(
            utab4.at[:, :, pl.ds(0, VBLK)], slots.at[k, 1], sems[k]).wait()
        ucol = jnp.broadcast_to(lax.bitwise_and(uv, jnp.int32(127)), (16,))
        pcol = jnp.broadcast_to(lax.bitwise_and(pv, jnp.int32(127)), (16,))
        icol = jnp.broadcast_to(i, (16,))
        u0 = plsc.load_gather(slots.at[k, 0], [s_lo, r_lo, ucol])
        u1 = plsc.load_gather(slots.at[k, 0], [s_hi, r_lo, ucol])
        p0 = plsc.load_gather(slots.at[k, 1], [s_lo, r_lo, pcol])
        p1 = plsc.load_gather(slots.at[k, 1], [s_hi, r_lo, pcol])
        plsc.store_scatter(ut_v, [d_lo, icol], u0)
        plsc.store_scatter(ut_v, [d_hi, icol], u1)
        plsc.store_scatter(pt_v, [d_lo, icol], p0)
        plsc.store_scatter(pt_v, [d_hi, icol], p1)

    def group(g, _):
        uvec = uidx_v[pl.ds(g * LANES, LANES)]
        pvec = pidx_v[pl.ds(g * LANES, LANES)]
        gm1 = lax.max(g - 1, 0)
        uvec_p = uidx_v[pl.ds(gm1 * LANES, LANES)]
        pvec_p = pidx_v[pl.ds(gm1 * LANES, LANES)]
        for j in range(LANES):
            k = j % NSLOT
            if j < NSLOT:
                @pl.when(g > 0)
                def _(j=j, k=k):
                    extract(k, uvec_p[LANES - NSLOT + j],
                            pvec_p[LANES - NSLOT + j],
                            (g - 1) * LANES + LANES - NSLOT + j)
            else:
                extract(k, uvec[j - NSLOT], pvec[j - NSLOT],
                        g * LANES + j - NSLOT)
            enq(k, uvec[j])
            enq_p(k, pvec[j])
        return 0

    lax.fori_loop(0, GROUPS, group, 0)

    uvec_t = uidx_v[pl.ds((GROUPS - 1) * LANES, LANES)]
    pvec_t = pidx_v[pl.ds((GROUPS - 1) * LANES, LANES)]
    for j in range(NSLOT):
        extract(j % NSLOT, uvec_t[LANES - NSLOT + j],
                pvec_t[LANES - NSLOT + j],
                (GROUPS - 1) * LANES + LANES - NSLOT + j)

    def dot(g, _):
        acc = jnp.zeros((16,), jnp.float32)
        for d in range(EMBED_DIM):
            acc = acc + (ut_v[d, pl.ds(g * LANES, LANES)]
                         * pt_v[d, pl.ds(g * LANES, LANES)])
        out_v[pl.ds(g * LANES, LANES)] = acc
        return 0

    lax.fori_loop(0, GROUPS, dot, 0)

    pltpu.sync_copy(out_v, out_hbm.at[pl.ds(base, B_PER_W)])


@jax.jit
def _sc_dot(uidx, pidx, utab4, ptab4):
    mesh = plsc.VectorSubcoreMesh(core_axis_name="c", subcore_axis_name="s")
    kern = pl.kernel(
        _body,
        out_type=jax.ShapeDtypeStruct((BATCH,), jnp.float32),
        mesh=mesh,
        scratch_types=[
            pltpu.VMEM((B_PER_W,), jnp.int32),
            pltpu.VMEM((B_PER_W,), jnp.int32),
            pltpu.VMEM((NSLOT, 2, 4, 8, VBLK), jnp.float32),
            pltpu.VMEM((EMBED_DIM, B_PER_W), jnp.float32),
            pltpu.VMEM((EMBED_DIM, B_PER_W), jnp.float32),
            pltpu.VMEM((B_PER_W,), jnp.float32),
            [pltpu.SemaphoreType.DMA] * NSLOT,
        ],
        compiler_params=pltpu.CompilerParams(
            needs_layout_passes=False, use_tc_tiling_on_sc=True),
    )
    return kern(uidx, pidx, utab4, ptab4)


def kernel(inputs, user_table, place_table):
    uidx = inputs[:, 0].astype(jnp.int32)
    pidx = inputs[:, 1].astype(jnp.int32)
    ut4 = user_table.T.reshape(4, 8, user_table.shape[0])
    pt4 = place_table.T.reshape(4, 8, place_table.shape[0])
    return _sc_dot(uidx, pidx, ut4, pt4)
